# Initial kernel scaffold; baseline (speedup 1.0000x reference)
import jax
import jax.numpy as jnp
from jax.experimental import pallas as pl


def kernel(patch_shape, index, position_embedding):
    raise NotImplementedError("write your pallas kernel here")


# speedup vs baseline: 2.7513x; 2.7513x over previous
import functools

import jax
import jax.numpy as jnp
from jax import lax
from jax.experimental import pallas as pl
from jax.experimental.pallas import tpu as pltpu
from jax.experimental.pallas import tpu_sc as plsc

D_MODEL = 128
NUM_WORKERS = 32
CHUNK = 128


@functools.partial(jax.jit, static_argnums=(0,))
def _gather_rows(n_chunks, index_w, table):
    rows_total = NUM_WORKERS * n_chunks * CHUNK
    rows_per_w = n_chunks * CHUNK
    mesh = plsc.VectorSubcoreMesh(core_axis_name="c", subcore_axis_name="s")

    @functools.partial(
        pl.kernel,
        mesh=mesh,
        out_type=jax.ShapeDtypeStruct((rows_total, D_MODEL), jnp.float32),
        scratch_types=[
            pltpu.VMEM((n_chunks, CHUNK), jnp.int32),
            pltpu.VMEM((CHUNK, D_MODEL), jnp.float32),
            pltpu.SemaphoreType.DMA,
        ],
    )
    def k(idx_hbm, table_hbm, out_hbm, idx_v, rows_v, gsem):
        wid = lax.axis_index("s") * 2 + lax.axis_index("c")
        base = wid * rows_per_w
        pltpu.sync_copy(idx_hbm.at[wid], idx_v)

        @pl.loop(0, n_chunks)
        def body(g):
            pltpu.async_copy(table_hbm.at[idx_v.at[g]], rows_v, gsem).wait()
            pltpu.sync_copy(rows_v, out_hbm.at[pl.ds(base + g * CHUNK, CHUNK)])

    return k(index_w, table)


def kernel(patch_shape, index, position_embedding):
    batch, patch_num = index.shape
    d_model = position_embedding.shape[1]
    rows = batch * patch_num
    n_chunks = rows // (NUM_WORKERS * CHUNK)
    idx_flat = index.astype(jnp.int32).reshape(NUM_WORKERS, n_chunks, CHUNK)
    out = _gather_rows(n_chunks, idx_flat, position_embedding)
    return out.reshape(batch, patch_num, d_model)


# speedup vs baseline: 2.9050x; 1.0559x over previous
import functools

import jax
import jax.numpy as jnp
from jax import lax
from jax.experimental import pallas as pl
from jax.experimental.pallas import tpu as pltpu
from jax.experimental.pallas import tpu_sc as plsc

D_MODEL = 128
NUM_WORKERS = 32
CHUNK = 128
NBUF = 4


@functools.partial(jax.jit, static_argnums=(0,))
def _gather_rows(n_chunks, index_w, table):
    rows_total = NUM_WORKERS * n_chunks * CHUNK
    rows_per_w = n_chunks * CHUNK
    mesh = plsc.VectorSubcoreMesh(core_axis_name="c", subcore_axis_name="s")
    assert n_chunks >= 8 and (n_chunks - 6) % NBUF == 0

    @functools.partial(
        pl.kernel,
        mesh=mesh,
        out_type=jax.ShapeDtypeStruct((rows_total, D_MODEL), jnp.float32),
        scratch_types=[
            pltpu.VMEM((n_chunks, CHUNK), jnp.int32),
            *[pltpu.VMEM((CHUNK, D_MODEL), jnp.float32) for _ in range(NBUF)],
            *[pltpu.SemaphoreType.DMA for _ in range(2 * NBUF)],
        ],
    )
    def k(idx_hbm, table_hbm, out_hbm, idx_v, b0, b1, b2, b3,
          g0, g1, g2, g3, w0, w1, w2, w3):
        bufs = (b0, b1, b2, b3)
        gsems = (g0, g1, g2, g3)
        wsems = (w0, w1, w2, w3)
        wid = lax.axis_index("s") * 2 + lax.axis_index("c")
        base = wid * rows_per_w
        pltpu.sync_copy(idx_hbm.at[wid], idx_v)

        def g_start(gg, bi):
            pltpu.async_copy(table_hbm.at[idx_v.at[gg]], bufs[bi], gsems[bi])

        def g_wait(bi):
            pltpu.make_async_copy(
                table_hbm.at[idx_v.at[0]], bufs[bi], gsems[bi]).wait()

        def w_start(gg, bi):
            pltpu.async_copy(
                bufs[bi], out_hbm.at[pl.ds(base + gg * CHUNK, CHUNK)],
                wsems[bi])

        def w_wait(bi):
            pltpu.make_async_copy(
                bufs[bi], out_hbm.at[pl.ds(base, CHUNK)], wsems[bi]).wait()

        g_start(0, 0)
        g_start(1, 1)
        for gg in (0, 1):
            g_wait(gg % NBUF)
            w_start(gg, gg % NBUF)
            g_start(gg + 2, (gg + 2) % NBUF)

        @pl.loop(2, n_chunks - NBUF, step=NBUF)
        def body(g):
            for b in range(NBUF):
                bi = (2 + b) % NBUF
                gg = g + b
                g_wait(bi)
                w_start(gg, bi)
                w_wait((bi + 2) % NBUF)
                g_start(gg + 2, (bi + 2) % NBUF)

        for gg in (n_chunks - 4, n_chunks - 3):
            g_wait(gg % NBUF)
            w_start(gg, gg % NBUF)
            w_wait((gg + 2) % NBUF)
            g_start(gg + 2, (gg + 2) % NBUF)
        for gg in (n_chunks - 2, n_chunks - 1):
            g_wait(gg % NBUF)
            w_start(gg, gg % NBUF)
            w_wait((gg + 2) % NBUF)
        for gg in (n_chunks - 2, n_chunks - 1):
            w_wait(gg % NBUF)

    return k(index_w, table)


def kernel(patch_shape, index, position_embedding):
    batch, patch_num = index.shape
    d_model = position_embedding.shape[1]
    rows = batch * patch_num
    n_chunks = rows // (NUM_WORKERS * CHUNK)
    idx_flat = index.astype(jnp.int32).reshape(NUM_WORKERS, n_chunks, CHUNK)
    out = _gather_rows(n_chunks, idx_flat, position_embedding)
    return out.reshape(batch, patch_num, d_model)


# speedup vs baseline: 4.5754x; 1.5750x over previous
import functools

import jax
import jax.numpy as jnp
from jax import lax
from jax.experimental import pallas as pl
from jax.experimental.pallas import tpu as pltpu
from jax.experimental.pallas import tpu_sc as plsc

NUM_WORKERS = 32
NBUF = 4


@functools.partial(jax.jit, static_argnums=())
def _gather_rows(index, table):
    batch, patch = index.shape
    d_model = table.shape[1]
    nb = batch // NUM_WORKERS
    mesh = plsc.VectorSubcoreMesh(core_axis_name="c", subcore_axis_name="s")
    assert nb >= 12 and (nb - 8) % NBUF == 0

    @functools.partial(
        pl.kernel,
        mesh=mesh,
        out_type=jax.ShapeDtypeStruct((batch, patch, d_model), jnp.float32),
        scratch_types=[
            pltpu.VMEM((nb, patch), jnp.int32),
            *[pltpu.VMEM((patch, d_model), jnp.float32) for _ in range(NBUF)],
            *[pltpu.SemaphoreType.DMA for _ in range(2 * NBUF)],
        ],
        compiler_params=pltpu.CompilerParams(use_tc_tiling_on_sc=True),
    )
    def k(idx_hbm, table_hbm, out_hbm, idx_v, b0, b1, b2, b3,
          g0, g1, g2, g3, w0, w1, w2, w3):
        bufs = (b0, b1, b2, b3)
        gsems = (g0, g1, g2, g3)
        wsems = (w0, w1, w2, w3)
        wid = lax.axis_index("s") * 2 + lax.axis_index("c")
        base = wid * nb
        pltpu.sync_copy(idx_hbm.at[pl.ds(base, nb)], idx_v)

        def g_start(gg, bi):
            pltpu.async_copy(table_hbm.at[idx_v.at[gg]], bufs[bi], gsems[bi])

        def g_wait(bi):
            pltpu.make_async_copy(
                table_hbm.at[idx_v.at[0]], bufs[bi], gsems[bi]).wait()

        def w_start(gg, bi):
            pltpu.async_copy(bufs[bi], out_hbm.at[base + gg], wsems[bi])

        def w_wait(bi):
            pltpu.make_async_copy(
                bufs[bi], out_hbm.at[base], wsems[bi]).wait()

        g_start(0, 0)
        g_start(1, 1)
        for gg in (0, 1):
            g_wait(gg % NBUF)
            w_start(gg, gg % NBUF)
            g_start(gg + 2, (gg + 2) % NBUF)

        @pl.loop(2, nb - 6, step=NBUF)
        def body(g):
            for b in range(NBUF):
                bi = (2 + b) % NBUF
                gg = g + b
                g_wait(bi)
                w_start(gg, bi)
                w_wait((bi + 2) % NBUF)
                g_start(gg + 2, (bi + 2) % NBUF)

        for gg in range(nb - 6, nb - 2):
            g_wait(gg % NBUF)
            w_start(gg, gg % NBUF)
            w_wait((gg + 2) % NBUF)
            g_start(gg + 2, (gg + 2) % NBUF)
        for gg in (nb - 2, nb - 1):
            g_wait(gg % NBUF)
            w_start(gg, gg % NBUF)
            w_wait((gg + 2) % NBUF)
        for gg in (nb - 2, nb - 1):
            w_wait(gg % NBUF)

    return k(index, table)


def kernel(patch_shape, index, position_embedding):
    return _gather_rows(index.astype(jnp.int32), position_embedding)


# speedup vs baseline: 7.1601x; 1.5649x over previous
import functools

import jax
import jax.numpy as jnp
from jax import lax
from jax.experimental import pallas as pl
from jax.experimental.pallas import tpu as pltpu
from jax.experimental.pallas import tpu_sc as plsc

D_MODEL = 128
NUM_WORKERS = 32
CHUNK = 128
NBUF = 4


@functools.partial(jax.jit, static_argnums=(0,))
def _gather_rows(n_chunks, index_w, table):
    rows_total = NUM_WORKERS * n_chunks * CHUNK
    rows_per_w = n_chunks * CHUNK
    mesh = plsc.VectorSubcoreMesh(core_axis_name="c", subcore_axis_name="s")
    assert n_chunks >= 10 and (n_chunks - 6) % NBUF == 0

    @functools.partial(
        pl.kernel,
        mesh=mesh,
        out_type=jax.ShapeDtypeStruct((rows_total, D_MODEL), jnp.float32),
        scratch_types=[
            pltpu.VMEM((n_chunks, CHUNK), jnp.int32),
            *[pltpu.VMEM((CHUNK, D_MODEL), jnp.float32) for _ in range(NBUF)],
            *[pltpu.SemaphoreType.DMA for _ in range(2 * NBUF)],
        ],
    )
    def k(idx_hbm, table_hbm, out_hbm, idx_v, b0, b1, b2, b3,
          g0, g1, g2, g3, w0, w1, w2, w3):
        bufs = (b0, b1, b2, b3)
        gsems = (g0, g1, g2, g3)
        wsems = (w0, w1, w2, w3)
        wid = lax.axis_index("s") * 2 + lax.axis_index("c")
        base = wid * rows_per_w
        pltpu.sync_copy(idx_hbm.at[wid], idx_v)

        def g_start(gg, bi):
            pltpu.async_copy(table_hbm.at[idx_v.at[gg]], bufs[bi], gsems[bi])

        def g_wait(bi):
            pltpu.make_async_copy(
                table_hbm.at[idx_v.at[0]], bufs[bi], gsems[bi]).wait()

        def w_start(gg, bi):
            pltpu.async_copy(
                bufs[bi], out_hbm.at[pl.ds(base + gg * CHUNK, CHUNK)],
                wsems[bi])

        def w_wait(bi):
            pltpu.make_async_copy(
                bufs[bi], out_hbm.at[pl.ds(base, CHUNK)], wsems[bi]).wait()

        g_start(0, 0)
        g_start(1, 1)
        for gg in (0, 1):
            g_wait(gg % NBUF)
            w_start(gg, gg % NBUF)
            g_start(gg + 2, (gg + 2) % NBUF)

        @pl.loop(2, n_chunks - 4, step=NBUF)
        def body(g):
            for b in range(NBUF):
                bi = (2 + b) % NBUF
                gg = g + b
                g_wait(bi)
                w_start(gg, bi)
                w_wait((bi + 2) % NBUF)
                g_start(gg + 2, (bi + 2) % NBUF)

        for gg in range(n_chunks - 4, n_chunks - 2):
            g_wait(gg % NBUF)
            w_start(gg, gg % NBUF)
            w_wait((gg + 2) % NBUF)
            g_start(gg + 2, (gg + 2) % NBUF)
        for gg in (n_chunks - 2, n_chunks - 1):
            g_wait(gg % NBUF)
            w_start(gg, gg % NBUF)
            w_wait((gg + 2) % NBUF)
        for gg in (n_chunks - 2, n_chunks - 1):
            w_wait(gg % NBUF)

    return k(index_w, table)


def kernel(patch_shape, index, position_embedding):
    batch, patch_num = index.shape
    d_model = position_embedding.shape[1]
    rows = batch * patch_num
    n_chunks = rows // (NUM_WORKERS * CHUNK)
    idx_t = index.astype(jnp.int32).T.reshape(NUM_WORKERS, n_chunks, CHUNK)
    out = _gather_rows(n_chunks, idx_t, position_embedding)
    return out.reshape(patch_num, batch, d_model).transpose(1, 0, 2)


# speedup vs baseline: 15.3910x; 2.1495x over previous
import functools

import jax
import jax.numpy as jnp
from jax import lax
from jax.experimental import pallas as pl
from jax.experimental.pallas import tpu as pltpu
from jax.experimental.pallas import tpu_sc as plsc

D_MODEL = 128
NUM_WORKERS = 32
CHUNK = 128
NBUF = 4


@functools.partial(jax.jit, static_argnums=(0,))
def _gather_rows(n_chunks, index_w, table):
    rows_total = NUM_WORKERS * n_chunks * CHUNK
    rows_per_w = n_chunks * CHUNK
    mesh = plsc.VectorSubcoreMesh(core_axis_name="c", subcore_axis_name="s")
    assert n_chunks >= 10 and (n_chunks - 6) % NBUF == 0

    @functools.partial(
        pl.kernel,
        mesh=mesh,
        out_type=jax.ShapeDtypeStruct((rows_total, D_MODEL), jnp.float32),
        scratch_types=[
            pltpu.VMEM((n_chunks, CHUNK), jnp.int32),
            pltpu.VMEM_SHARED((1000, D_MODEL), jnp.float32),
            *[pltpu.VMEM((CHUNK, D_MODEL), jnp.float32) for _ in range(NBUF)],
            *[pltpu.SemaphoreType.DMA for _ in range(2 * NBUF)],
        ],
    )
    def k(idx_hbm, table_hbm, out_hbm, idx_v, table_sp, b0, b1, b2, b3,
          g0, g1, g2, g3, w0, w1, w2, w3):
        bufs = (b0, b1, b2, b3)
        gsems = (g0, g1, g2, g3)
        wsems = (w0, w1, w2, w3)
        wid = lax.axis_index("s") * 2 + lax.axis_index("c")
        base = wid * rows_per_w

        @pl.when(lax.axis_index("s") == 0)
        def _():
            pltpu.sync_copy(table_hbm, table_sp)

        pltpu.sync_copy(idx_hbm.at[wid], idx_v)
        plsc.subcore_barrier()

        def g_start(gg, bi):
            pltpu.async_copy(table_sp.at[idx_v.at[gg]], bufs[bi], gsems[bi])

        def g_wait(bi):
            pltpu.make_async_copy(
                table_sp.at[idx_v.at[0]], bufs[bi], gsems[bi]).wait()

        def w_start(gg, bi):
            pltpu.async_copy(
                bufs[bi], out_hbm.at[pl.ds(base + gg * CHUNK, CHUNK)],
                wsems[bi])

        def w_wait(bi):
            pltpu.make_async_copy(
                bufs[bi], out_hbm.at[pl.ds(base, CHUNK)], wsems[bi]).wait()

        g_start(0, 0)
        g_start(1, 1)
        for gg in (0, 1):
            g_wait(gg % NBUF)
            w_start(gg, gg % NBUF)
            g_start(gg + 2, (gg + 2) % NBUF)

        @pl.loop(2, n_chunks - 4, step=NBUF)
        def body(g):
            for b in range(NBUF):
                bi = (2 + b) % NBUF
                gg = g + b
                g_wait(bi)
                w_start(gg, bi)
                w_wait((bi + 2) % NBUF)
                g_start(gg + 2, (bi + 2) % NBUF)

        for gg in range(n_chunks - 4, n_chunks - 2):
            g_wait(gg % NBUF)
            w_start(gg, gg % NBUF)
            w_wait((gg + 2) % NBUF)
            g_start(gg + 2, (gg + 2) % NBUF)
        for gg in (n_chunks - 2, n_chunks - 1):
            g_wait(gg % NBUF)
            w_start(gg, gg % NBUF)
            w_wait((gg + 2) % NBUF)
        for gg in (n_chunks - 2, n_chunks - 1):
            w_wait(gg % NBUF)

    return k(index_w, table)


def kernel(patch_shape, index, position_embedding):
    batch, patch_num = index.shape
    d_model = position_embedding.shape[1]
    rows = batch * patch_num
    n_chunks = rows // (NUM_WORKERS * CHUNK)
    idx_t = index.astype(jnp.int32).T.reshape(NUM_WORKERS, n_chunks, CHUNK)
    out = _gather_rows(n_chunks, idx_t, position_embedding)
    return out.reshape(patch_num, batch, d_model).transpose(1, 0, 2)
